# baseline (speedup 1.0000x reference)
import functools

import jax

jax.config.update("jax_enable_x64", True)
import jax.numpy as jnp
import numpy as np
from jax import lax
from jax.experimental import pallas as pl
from jax.experimental.pallas import tpu as pltpu
from jax.experimental.pallas import tpu_sc as plsc

K = 4
ALPHA = 0.1
BETA = 0.9
N = 10000
E = 320000
D = 128

NC, NS = 2, 16
NP = 10240
C = 128
CPT = 80
HCPT = CPT // 2
EW = C * CPT
EPAD = EW * NC * NS
RPT = NP // NS

_mesh = plsc.VectorSubcoreMesh(core_axis_name="c", subcore_axis_name="s")


def _make_scatter():
    @functools.partial(
        pl.kernel,
        out_type=jax.ShapeDtypeStruct((NC * NP, D), jnp.float32),
        mesh=_mesh,
        scratch_types=[
            pltpu.VMEM((2 * HCPT, C), jnp.int32),
            pltpu.VMEM((C, D), jnp.float32),
            pltpu.VMEM((C, D), jnp.float32),
            pltpu.VMEM_SHARED((NP, D), jnp.float32),
            pltpu.SemaphoreType.DMA,
            pltpu.SemaphoreType.DMA,
        ],
    )
    def scatter_sum(u_hbm, ep_hbm, zeros_hbm, out_hbm, epb, ra, rb, acc,
                    sa, sb):
        c = lax.axis_index("c")
        s = lax.axis_index("s")
        w = c * NS + s
        i0 = jnp.int32(0)
        pltpu.sync_copy(zeros_hbm, acc.at[pl.ds(s * RPT, RPT)])
        pltpu.sync_copy(ep_hbm.at[pl.ds(w * 2 * CPT, 2 * HCPT)], epb)
        plsc.subcore_barrier()

        def run_half():
            pltpu.async_copy(u_hbm.at[epb.at[i0]], ra, sa)

            def pair(j, carry):
                i = j * 2
                pltpu.make_async_copy(u_hbm.at[epb.at[i * 2]], ra, sa).wait()
                pltpu.async_copy(u_hbm.at[epb.at[i * 2 + 2]], rb, sb)
                pltpu.sync_copy(ra, acc.at[epb.at[i * 2 + 1]], add=True)
                pltpu.make_async_copy(
                    u_hbm.at[epb.at[i * 2 + 2]], rb, sb).wait()
                pltpu.async_copy(u_hbm.at[epb.at[i * 2 + 4]], ra, sa)
                pltpu.sync_copy(rb, acc.at[epb.at[i * 2 + 3]], add=True)
                return carry

            lax.fori_loop(i0, jnp.int32(HCPT // 2 - 1), pair, i0)
            i = jnp.int32(HCPT - 2)
            pltpu.make_async_copy(u_hbm.at[epb.at[i * 2]], ra, sa).wait()
            pltpu.async_copy(u_hbm.at[epb.at[i * 2 + 2]], rb, sb)
            pltpu.sync_copy(ra, acc.at[epb.at[i * 2 + 1]], add=True)
            pltpu.make_async_copy(u_hbm.at[epb.at[i * 2 + 2]], rb, sb).wait()
            pltpu.sync_copy(rb, acc.at[epb.at[i * 2 + 3]], add=True)

        run_half()
        pltpu.sync_copy(
            ep_hbm.at[pl.ds(w * 2 * CPT + 2 * HCPT, 2 * HCPT)], epb)
        run_half()
        plsc.subcore_barrier()
        pltpu.sync_copy(acc.at[pl.ds(s * RPT, RPT)],
                        out_hbm.at[pl.ds(c * NP + s * RPT, RPT)])

    return scatter_sum


_scatter_sum = _make_scatter()


_BR = 1024


def _ew_call(body, n_in):
    zero = np.int32(0)
    specs = [pl.BlockSpec((_BR, D), lambda i: (i, zero)) for _ in range(n_in)]
    return pl.pallas_call(
        body,
        grid=(NP // _BR,),
        in_specs=specs,
        out_specs=pl.BlockSpec((_BR, D), lambda i: (i, np.int32(0))),
        out_shape=jax.ShapeDtypeStruct((NP, D), jnp.float32),
    )


def _setup_body(d0, d1, x, norm_o, n2_o, u0_o):
    nm = lax.rsqrt(d0[...] + d1[...])
    norm_o[...] = nm
    n2_o[...] = nm * nm
    u0_o[...] = nm * x[...]


def _setup(d0, d1, x_pad):
    zero = np.int32(0)
    spec = pl.BlockSpec((_BR, D), lambda i: (i, zero))
    return pl.pallas_call(
        _setup_body,
        grid=(NP // _BR,),
        in_specs=[spec] * 3,
        out_specs=[spec] * 3,
        out_shape=[jax.ShapeDtypeStruct((NP, D), jnp.float32)] * 3,
    )(d0, d1, x_pad)


def _combine(ca, cb, scale, p0, p1, base):
    ca = float(ca)
    cb = float(cb)

    def body(sc, a0, a1, b, out):
        out[...] = ca * sc[...] * (a0[...] + a1[...]) + cb * b[...]

    return _ew_call(body, 4)(scale, p0, p1, base)


def kernel(x, edge_index):
    x = x.astype(jnp.float32)
    ei = edge_index.astype(jnp.int32)
    ei = ei[:, jnp.argsort(ei[0])]
    pad = jnp.full((1, EPAD - E), NP - 1, jnp.int32)
    ep = jnp.concatenate([ei, jnp.broadcast_to(pad, (2, EPAD - E))], axis=1)
    ep = ep.reshape(2, EPAD // C, C).transpose(1, 0, 2).reshape(-1, C)
    x_pad = jnp.zeros((NP, D), jnp.float32).at[:N, :].set(x)
    ones = jnp.ones((NP, D), jnp.float32)
    zeros_blk = jnp.zeros((RPT, D), jnp.float32)

    degp = _scatter_sum(ones, ep, zeros_blk)
    norm, n2, u0 = _setup(degp[:NP], degp[NP:], x_pad)

    gamma = float(np.power(BETA, K) + ALPHA * np.sum([np.power(BETA, i) for i in range(K)]))

    u = u0
    for _ in range(K - 1):
        p = _scatter_sum(u, ep, zeros_blk)
        u = _combine(BETA, ALPHA, n2, p[:NP], p[NP:], u0)
    p = _scatter_sum(u, ep, zeros_blk)
    h = _combine(BETA / gamma, ALPHA / gamma, norm, p[:NP], p[NP:], x_pad)
    return h[:N]


# speedup vs baseline: 1.1850x; 1.1850x over previous
import functools

import jax

jax.config.update("jax_enable_x64", True)
import jax.numpy as jnp
import numpy as np
from jax import lax
from jax.experimental import pallas as pl
from jax.experimental.pallas import tpu as pltpu
from jax.experimental.pallas import tpu_sc as plsc

K = 4
ALPHA = 0.1
BETA = 0.9
N = 10000
E = 320000
D = 128

NC, NS = 2, 16
NP = 10240
C = 128
CPT = 80
HCPT = CPT // 2
EW = C * CPT
EPAD = EW * NC * NS
RPT = NP // NS

_mesh = plsc.VectorSubcoreMesh(core_axis_name="c", subcore_axis_name="s")


def _make_scatter():
    @functools.partial(
        pl.kernel,
        out_type=jax.ShapeDtypeStruct((NC * NP, D), jnp.float32),
        mesh=_mesh,
        scratch_types=[
            pltpu.VMEM((2 * HCPT, C), jnp.int32),
            pltpu.VMEM((C, D), jnp.float32),
            pltpu.VMEM((C, D), jnp.float32),
            pltpu.VMEM_SHARED((NP, D), jnp.float32),
            pltpu.SemaphoreType.DMA,
            pltpu.SemaphoreType.DMA,
        ],
    )
    def scatter_sum(u_hbm, ep_hbm, zeros_hbm, out_hbm, epb, ra, rb, acc,
                    sa, sb):
        c = lax.axis_index("c")
        s = lax.axis_index("s")
        w = c * NS + s
        i0 = jnp.int32(0)
        pltpu.sync_copy(zeros_hbm, acc.at[pl.ds(s * RPT, RPT)])
        pltpu.sync_copy(ep_hbm.at[pl.ds(w * 2 * CPT, 2 * HCPT)], epb)
        plsc.subcore_barrier()

        HC = C // 2

        def gather2(row, buf, sem):
            pltpu.async_copy(u_hbm.at[epb.at[row, pl.ds(0, HC)]],
                             buf.at[pl.ds(0, HC)], sem)
            pltpu.async_copy(u_hbm.at[epb.at[row, pl.ds(HC, HC)]],
                             buf.at[pl.ds(HC, HC)], sem)

        def wait_full(buf, sem):
            pltpu.make_async_copy(u_hbm.at[epb.at[i0]], buf, sem).wait()

        def run_half():
            gather2(i0, ra, sa)

            def pair(j, carry):
                i = j * 2
                wait_full(ra, sa)
                gather2(i * 2 + 2, rb, sb)
                pltpu.sync_copy(ra, acc.at[epb.at[i * 2 + 1]], add=True)
                wait_full(rb, sb)
                gather2(i * 2 + 4, ra, sa)
                pltpu.sync_copy(rb, acc.at[epb.at[i * 2 + 3]], add=True)
                return carry

            lax.fori_loop(i0, jnp.int32(HCPT // 2 - 1), pair, i0)
            i = jnp.int32(HCPT - 2)
            wait_full(ra, sa)
            gather2(i * 2 + 2, rb, sb)
            pltpu.sync_copy(ra, acc.at[epb.at[i * 2 + 1]], add=True)
            wait_full(rb, sb)
            pltpu.sync_copy(rb, acc.at[epb.at[i * 2 + 3]], add=True)

        run_half()
        pltpu.sync_copy(
            ep_hbm.at[pl.ds(w * 2 * CPT + 2 * HCPT, 2 * HCPT)], epb)
        run_half()
        plsc.subcore_barrier()
        pltpu.sync_copy(acc.at[pl.ds(s * RPT, RPT)],
                        out_hbm.at[pl.ds(c * NP + s * RPT, RPT)])

    return scatter_sum


_scatter_sum = _make_scatter()


_BR = 1024


def _ew_call(body, n_in):
    zero = np.int32(0)
    specs = [pl.BlockSpec((_BR, D), lambda i: (i, zero)) for _ in range(n_in)]
    return pl.pallas_call(
        body,
        grid=(NP // _BR,),
        in_specs=specs,
        out_specs=pl.BlockSpec((_BR, D), lambda i: (i, np.int32(0))),
        out_shape=jax.ShapeDtypeStruct((NP, D), jnp.float32),
    )


def _setup_body(d0, d1, x, norm_o, n2_o, u0_o):
    nm = lax.rsqrt(d0[...] + d1[...])
    norm_o[...] = nm
    n2_o[...] = nm * nm
    u0_o[...] = nm * x[...]


def _setup(d0, d1, x_pad):
    zero = np.int32(0)
    spec = pl.BlockSpec((_BR, D), lambda i: (i, zero))
    return pl.pallas_call(
        _setup_body,
        grid=(NP // _BR,),
        in_specs=[spec] * 3,
        out_specs=[spec] * 3,
        out_shape=[jax.ShapeDtypeStruct((NP, D), jnp.float32)] * 3,
    )(d0, d1, x_pad)


def _combine(ca, cb, scale, p0, p1, base):
    ca = float(ca)
    cb = float(cb)

    def body(sc, a0, a1, b, out):
        out[...] = ca * sc[...] * (a0[...] + a1[...]) + cb * b[...]

    return _ew_call(body, 4)(scale, p0, p1, base)


def kernel(x, edge_index):
    x = x.astype(jnp.float32)
    ei = edge_index.astype(jnp.int32)
    pad = jnp.full((1, EPAD - E), NP - 1, jnp.int32)
    ep = jnp.concatenate([ei, jnp.broadcast_to(pad, (2, EPAD - E))], axis=1)
    ep = ep.reshape(2, EPAD // C, C).transpose(1, 0, 2).reshape(-1, C)
    x_pad = jnp.zeros((NP, D), jnp.float32).at[:N, :].set(x)
    ones = jnp.ones((NP, D), jnp.float32)
    zeros_blk = jnp.zeros((RPT, D), jnp.float32)

    degp = _scatter_sum(ones, ep, zeros_blk)
    norm, n2, u0 = _setup(degp[:NP], degp[NP:], x_pad)

    gamma = float(np.power(BETA, K) + ALPHA * np.sum([np.power(BETA, i) for i in range(K)]))

    u = u0
    for _ in range(K - 1):
        p = _scatter_sum(u, ep, zeros_blk)
        u = _combine(BETA, ALPHA, n2, p[:NP], p[NP:], u0)
    p = _scatter_sum(u, ep, zeros_blk)
    h = _combine(BETA / gamma, ALPHA / gamma, norm, p[:NP], p[NP:], x_pad)
    return h[:N]


# speedup vs baseline: 1.4721x; 1.2423x over previous
import functools

import jax

jax.config.update("jax_enable_x64", True)
import jax.numpy as jnp
import numpy as np
from jax import lax
from jax.experimental import pallas as pl
from jax.experimental.pallas import tpu as pltpu
from jax.experimental.pallas import tpu_sc as plsc

K = 4
ALPHA = 0.1
BETA = 0.9
N = 10000
E = 320000
D = 128

NC, NS = 2, 16
NP = 10240
C = 128
CPT = 80
HCPT = CPT // 2
EW = C * CPT
EPAD = EW * NC * NS
RPT = NP // NS

_mesh = plsc.VectorSubcoreMesh(core_axis_name="c", subcore_axis_name="s")


def _make_scatter():
    @functools.partial(
        pl.kernel,
        out_type=jax.ShapeDtypeStruct((NC * NP, D), jnp.float32),
        mesh=_mesh,
        scratch_types=[
            pltpu.VMEM((2 * HCPT, C), jnp.int32),
            pltpu.VMEM((C, D), jnp.float32),
            pltpu.VMEM((C, D), jnp.float32),
            pltpu.VMEM((16,), jnp.int32),
            pltpu.VMEM_SHARED((NP, D), jnp.float32),
            pltpu.SemaphoreType.DMA,
            pltpu.SemaphoreType.DMA,
        ],
    )
    def scatter_sum(u_hbm, ep_hbm, zeros_hbm, ones_hbm, mode_hbm, out_hbm,
                    epb, ra, rb, modev, acc, sa, sb):
        c = lax.axis_index("c")
        s = lax.axis_index("s")
        w = c * NS + s
        i0 = jnp.int32(0)
        pltpu.sync_copy(zeros_hbm, acc.at[pl.ds(s * RPT, RPT)])
        pltpu.sync_copy(ep_hbm.at[pl.ds(w * 2 * CPT, 2 * HCPT)], epb)
        pltpu.sync_copy(mode_hbm, modev)
        mode = modev[...][0]
        plsc.subcore_barrier()

        HC = C // 2

        def gather2(row, buf, sem):
            pltpu.async_copy(u_hbm.at[epb.at[row, pl.ds(0, HC)]],
                             buf.at[pl.ds(0, HC)], sem)
            pltpu.async_copy(u_hbm.at[epb.at[row, pl.ds(HC, HC)]],
                             buf.at[pl.ds(HC, HC)], sem)

        def wait_full(buf, sem):
            pltpu.make_async_copy(u_hbm.at[epb.at[i0]], buf, sem).wait()

        def run_half():
            gather2(i0, ra, sa)

            def pair(j, carry):
                i = j * 2
                wait_full(ra, sa)
                gather2(i * 2 + 2, rb, sb)
                pltpu.sync_copy(ra, acc.at[epb.at[i * 2 + 1]], add=True)
                wait_full(rb, sb)
                gather2(i * 2 + 4, ra, sa)
                pltpu.sync_copy(rb, acc.at[epb.at[i * 2 + 3]], add=True)
                return carry

            lax.fori_loop(i0, jnp.int32(HCPT // 2 - 1), pair, i0)
            i = jnp.int32(HCPT - 2)
            wait_full(ra, sa)
            gather2(i * 2 + 2, rb, sb)
            pltpu.sync_copy(ra, acc.at[epb.at[i * 2 + 1]], add=True)
            wait_full(rb, sb)
            pltpu.sync_copy(rb, acc.at[epb.at[i * 2 + 3]], add=True)

        @pl.when(mode == 1)
        def _gather_mode():
            run_half()
            pltpu.sync_copy(
                ep_hbm.at[pl.ds(w * 2 * CPT + 2 * HCPT, 2 * HCPT)], epb)
            run_half()

        @pl.when(mode == 0)
        def _degree_mode():
            pltpu.sync_copy(ones_hbm, ra)

            def chunk(i, carry):
                pltpu.sync_copy(ra, acc.at[epb.at[i * 2 + 1]], add=True)
                return carry

            lax.fori_loop(i0, jnp.int32(HCPT), chunk, i0)
            pltpu.sync_copy(
                ep_hbm.at[pl.ds(w * 2 * CPT + 2 * HCPT, 2 * HCPT)], epb)
            lax.fori_loop(i0, jnp.int32(HCPT), chunk, i0)

        plsc.subcore_barrier()
        pltpu.sync_copy(acc.at[pl.ds(s * RPT, RPT)],
                        out_hbm.at[pl.ds(c * NP + s * RPT, RPT)])

    return scatter_sum


_scatter_sum = _make_scatter()


_BR = 1024


def _ew_call(body, n_in):
    zero = np.int32(0)
    specs = [pl.BlockSpec((_BR, D), lambda i: (i, zero)) for _ in range(n_in)]
    return pl.pallas_call(
        body,
        grid=(NP // _BR,),
        in_specs=specs,
        out_specs=pl.BlockSpec((_BR, D), lambda i: (i, np.int32(0))),
        out_shape=jax.ShapeDtypeStruct((NP, D), jnp.float32),
    )


def _setup_body(d0, d1, x, norm_o, n2_o, u0_o):
    nm = lax.rsqrt(d0[...] + d1[...])
    norm_o[...] = nm
    n2_o[...] = nm * nm
    u0_o[...] = nm * x[...]


def _setup(d0, d1, x_pad):
    zero = np.int32(0)
    spec = pl.BlockSpec((_BR, D), lambda i: (i, zero))
    return pl.pallas_call(
        _setup_body,
        grid=(NP // _BR,),
        in_specs=[spec] * 3,
        out_specs=[spec] * 3,
        out_shape=[jax.ShapeDtypeStruct((NP, D), jnp.float32)] * 3,
    )(d0, d1, x_pad)


def _combine(ca, cb, scale, p0, p1, base):
    ca = float(ca)
    cb = float(cb)

    def body(sc, a0, a1, b, out):
        out[...] = ca * sc[...] * (a0[...] + a1[...]) + cb * b[...]

    return _ew_call(body, 4)(scale, p0, p1, base)


def kernel(x, edge_index):
    x = x.astype(jnp.float32)
    ei = edge_index.astype(jnp.int32)
    pad = jnp.full((1, EPAD - E), NP - 1, jnp.int32)
    ep = jnp.concatenate([ei, jnp.broadcast_to(pad, (2, EPAD - E))], axis=1)
    ep = ep.reshape(2, EPAD // C, C).transpose(1, 0, 2).reshape(-1, C)
    x_pad = jnp.zeros((NP, D), jnp.float32).at[:N, :].set(x)
    ones_blk = jnp.ones((C, D), jnp.float32)
    zeros_blk = jnp.zeros((RPT, D), jnp.float32)
    m_deg = jnp.zeros((16,), jnp.int32)
    m_gat = jnp.ones((16,), jnp.int32)

    degp = _scatter_sum(x_pad, ep, zeros_blk, ones_blk, m_deg)
    norm, n2, u0 = _setup(degp[:NP], degp[NP:], x_pad)

    gamma = float(np.power(BETA, K) + ALPHA * np.sum([np.power(BETA, i) for i in range(K)]))

    u = u0
    for _ in range(K - 1):
        p = _scatter_sum(u, ep, zeros_blk, ones_blk, m_gat)
        u = _combine(BETA, ALPHA, n2, p[:NP], p[NP:], u0)
    p = _scatter_sum(u, ep, zeros_blk, ones_blk, m_gat)
    h = _combine(BETA / gamma, ALPHA / gamma, norm, p[:NP], p[NP:], x_pad)
    return h[:N]
